# Initial kernel scaffold; baseline (speedup 1.0000x reference)
import jax
import jax.numpy as jnp
from jax.experimental import pallas as pl


def kernel(x, weights):
    raise NotImplementedError("write your pallas kernel here")


# speedup vs baseline: 7.5631x; 7.5631x over previous
import functools

import jax
import jax.numpy as jnp
from jax import lax
from jax.experimental import pallas as pl
from jax.experimental.pallas import tpu as pltpu
from jax.experimental.pallas import tpu_sc as plsc

L = 16
NC = 2
NS = 16


@functools.lru_cache(maxsize=None)
def _make_bmu_kernel(B: int, N: int, D: int):
    nchunk = N // L
    mesh = plsc.VectorSubcoreMesh(core_axis_name="c", subcore_axis_name="s")

    @functools.partial(
        pl.kernel,
        mesh=mesh,
        out_type=jax.ShapeDtypeStruct((B * D,), jnp.float32),
        compiler_params=pltpu.CompilerParams(needs_layout_passes=False),
        scratch_types=[
            pltpu.VMEM((D,), jnp.float32),
            pltpu.VMEM((N * D,), jnp.float32),
            pltpu.VMEM((D,), jnp.float32),
        ],
    )
    def bmu(x_hbm, wt_hbm, out_hbm, x_v, wt_v, row_v):
        wid = lax.axis_index("s") * NC + lax.axis_index("c")

        @pl.when(wid < B)
        def _():
            pltpu.sync_copy(x_hbm.at[pl.ds(wid * D, D)], x_v)
            pltpu.sync_copy(wt_hbm, wt_v)
            iota = lax.iota(jnp.int32, L)
            best_val = jnp.full((L,), jnp.inf, jnp.float32)
            best_idx = jnp.zeros((L,), jnp.int32)
            for c in range(nchunk):
                def body(d, acc):
                    xd = plsc.load_gather(x_v, [jnp.full((L,), d, jnp.int32)])
                    wv = wt_v[pl.ds(d * N + c * L, L)]
                    df = xd - wv
                    return acc + df * df
                acc = lax.fori_loop(0, D, body, jnp.zeros((L,), jnp.float32))
                m = acc < best_val
                best_val = jnp.where(m, acc, best_val)
                best_idx = jnp.where(m, c * L + iota, best_idx)
            mval = jnp.min(best_val)
            cand = jnp.where(best_val == mval, best_idx, jnp.int32(2**30))
            j = jnp.min(cand)
            for k in range(D // L):
                row_v[pl.ds(k * L, L)] = plsc.load_gather(
                    wt_v, [(iota + k * L) * N + j])
            pltpu.sync_copy(row_v, out_hbm.at[pl.ds(wid * D, D)])

    return bmu


def kernel(x, weights):
    b, _, f = x.shape
    n = weights.shape[0]
    xq = x[:, -1, :].reshape(b * f)
    wt = weights.T.reshape(n * f)
    out = _make_bmu_kernel(b, n, f)(xq, wt)
    return out.reshape(b, f)


# speedup vs baseline: 8.1711x; 1.0804x over previous
import functools

import jax
import jax.numpy as jnp
from jax import lax
from jax.experimental import pallas as pl
from jax.experimental.pallas import tpu as pltpu
from jax.experimental.pallas import tpu_sc as plsc

L = 16
NQ = 4
NS = 16


@functools.lru_cache(maxsize=None)
def _make_bmu_kernel(B: int, N: int, D: int):
    n_per_w = N // NQ
    nchunk = n_per_w // L
    mesh = plsc.VectorSubcoreMesh(core_axis_name="c", subcore_axis_name="s")

    @functools.partial(
        pl.kernel,
        mesh=mesh,
        out_type=jax.ShapeDtypeStruct((B * D,), jnp.float32),
        compiler_params=pltpu.CompilerParams(needs_layout_passes=False),
        scratch_types=[
            pltpu.VMEM((D,), jnp.float32),
            pltpu.VMEM((n_per_w, D), jnp.float32),
            pltpu.VMEM((L,), jnp.float32),
            pltpu.VMEM((NQ, L), jnp.float32),
            pltpu.VMEM_SHARED((NS, L), jnp.float32),
        ],
    )
    def bmu(x_hbm, w_hbm, out_hbm, x_v, w_v, pub_v, mins_v, board):
        c = lax.axis_index("c")
        s = lax.axis_index("s")
        b = c * (NS // NQ) + s // NQ
        q = s % NQ

        pltpu.sync_copy(x_hbm.at[pl.ds(b * D, D)], x_v)
        pltpu.sync_copy(w_hbm.at[pl.ds(q * n_per_w, n_per_w)], w_v)

        iota = lax.iota(jnp.int32, L)
        accs = [jnp.zeros((L,), jnp.float32) for _ in range(nchunk)]
        for d in range(D):
            dvec = jnp.full((L,), d, jnp.int32)
            xd = plsc.load_gather(x_v, [dvec])
            for cc in range(nchunk):
                wv = plsc.load_gather(w_v, [cc * L + iota, dvec])
                df = xd - wv
                accs[cc] = accs[cc] + df * df

        best_val = accs[0]
        best_idx = iota
        for cc in range(1, nchunk):
            m = accs[cc] < best_val
            best_val = jnp.where(m, accs[cc], best_val)
            best_idx = jnp.where(m, cc * L + iota, best_idx)
        mval = jnp.min(best_val)
        cand = jnp.where(best_val == mval, best_idx, jnp.int32(2**30))
        j_loc = jnp.min(cand)

        pub_v[...] = jnp.full((L,), mval, jnp.float32)
        pltpu.sync_copy(pub_v, board.at[s])
        plsc.subcore_barrier()

        pltpu.sync_copy(board.at[pl.ds((s // NQ) * NQ, NQ)], mins_v)
        m0 = jnp.min(mins_v[0])
        m1 = jnp.min(mins_v[1])
        m2 = jnp.min(mins_v[2])
        m3 = jnp.min(mins_v[3])
        gmin = jnp.minimum(jnp.minimum(m0, m1), jnp.minimum(m2, m3))
        first_q = jnp.where(
            m0 == gmin, 0,
            jnp.where(m1 == gmin, 1, jnp.where(m2 == gmin, 2, 3)))

        @pl.when(q == first_q)
        def _():
            pltpu.sync_copy(w_v.at[j_loc], out_hbm.at[pl.ds(b * D, D)])

    return bmu


def kernel(x, weights):
    b, _, f = x.shape
    n = weights.shape[0]
    xq = x[:, -1, :].reshape(b * f)
    out = _make_bmu_kernel(b, n, f)(xq, weights)
    return out.reshape(b, f)


# speedup vs baseline: 9.5575x; 1.1697x over previous
import functools

import jax
import jax.numpy as jnp
from jax import lax
from jax.experimental import pallas as pl
from jax.experimental.pallas import tpu as pltpu
from jax.experimental.pallas import tpu_sc as plsc

L = 16
NQ = 4
NS = 16


@functools.lru_cache(maxsize=None)
def _make_bmu_kernel(B: int, N: int, D: int):
    n_per_w = N // NQ
    nchunk = n_per_w // L
    mesh = plsc.VectorSubcoreMesh(core_axis_name="c", subcore_axis_name="s")

    @functools.partial(
        pl.kernel,
        mesh=mesh,
        out_type=jax.ShapeDtypeStruct((B * D,), jnp.float32),
        compiler_params=pltpu.CompilerParams(needs_layout_passes=False),
        scratch_types=[
            pltpu.VMEM((D,), jnp.float32),
            pltpu.VMEM((D * n_per_w,), jnp.float32),
            pltpu.VMEM((L,), jnp.float32),
            pltpu.VMEM((NQ * L,), jnp.float32),
            pltpu.VMEM((D,), jnp.float32),
            pltpu.VMEM_SHARED((NS * L,), jnp.float32),
        ],
    )
    def bmu(x_hbm, wq_hbm, out_hbm, x_v, w_v, pub_v, mins_v, row_v, board):
        c = lax.axis_index("c")
        s = lax.axis_index("s")
        b = c * (NS // NQ) + s // NQ
        q = s % NQ

        pltpu.sync_copy(x_hbm.at[pl.ds(b * D, D)], x_v)
        pltpu.sync_copy(wq_hbm.at[pl.ds(q * D * n_per_w, D * n_per_w)], w_v)

        iota = lax.iota(jnp.int32, L)
        best_val = jnp.full((L,), jnp.inf, jnp.float32)
        best_idx = jnp.zeros((L,), jnp.int32)
        for cc in range(nchunk):
            def body(d, acc):
                xd = plsc.load_gather(x_v, [jnp.full((L,), d, jnp.int32)])
                wv = w_v[pl.ds(d * n_per_w + cc * L, L)]
                df = xd - wv
                return acc + df * df
            acc = lax.fori_loop(0, D, body, jnp.zeros((L,), jnp.float32))
            m = acc < best_val
            best_val = jnp.where(m, acc, best_val)
            best_idx = jnp.where(m, cc * L + iota, best_idx)
        mval = jnp.min(best_val)
        cand = jnp.where(best_val == mval, best_idx, jnp.int32(2**30))
        j_loc = jnp.min(cand)

        pub_v[...] = jnp.full((L,), mval, jnp.float32)
        pltpu.sync_copy(pub_v, board.at[pl.ds(s * L, L)])
        plsc.subcore_barrier()

        pltpu.sync_copy(board.at[pl.ds((s // NQ) * NQ * L, NQ * L)], mins_v)
        m0 = jnp.min(mins_v[pl.ds(0 * L, L)])
        m1 = jnp.min(mins_v[pl.ds(1 * L, L)])
        m2 = jnp.min(mins_v[pl.ds(2 * L, L)])
        m3 = jnp.min(mins_v[pl.ds(3 * L, L)])
        gmin = jnp.minimum(jnp.minimum(m0, m1), jnp.minimum(m2, m3))
        first_q = jnp.where(
            m0 == gmin, 0,
            jnp.where(m1 == gmin, 1, jnp.where(m2 == gmin, 2, 3)))

        @pl.when(q == first_q)
        def _():
            for k in range(D // L):
                row_v[pl.ds(k * L, L)] = plsc.load_gather(
                    w_v, [(iota + k * L) * n_per_w + j_loc])
            pltpu.sync_copy(row_v, out_hbm.at[pl.ds(b * D, D)])

    return bmu


def kernel(x, weights):
    b, _, f = x.shape
    n = weights.shape[0]
    xq = x[:, -1, :].reshape(b * f)
    wq = weights.T.reshape(f, NQ, n // NQ).transpose(1, 0, 2).reshape(-1)
    out = _make_bmu_kernel(b, n, f)(xq, wq)
    return out.reshape(b, f)


# speedup vs baseline: 10.2822x; 1.0758x over previous
import functools

import jax
import jax.numpy as jnp
from jax import lax
from jax.experimental import pallas as pl
from jax.experimental.pallas import tpu as pltpu
from jax.experimental.pallas import tpu_sc as plsc

L = 16


@functools.lru_cache(maxsize=None)
def _make_bmu_kernel(B: int, N: int, D: int):
    nchunk = N // L
    mesh = plsc.VectorSubcoreMesh(
        core_axis_name="c", subcore_axis_name="s", num_cores=1)

    @functools.partial(
        pl.kernel,
        mesh=mesh,
        out_type=jax.ShapeDtypeStruct((B * D,), jnp.float32),
        compiler_params=pltpu.CompilerParams(needs_layout_passes=False),
        scratch_types=[
            pltpu.VMEM((D,), jnp.float32),
            pltpu.VMEM((D * N,), jnp.float32),
            pltpu.VMEM((D,), jnp.float32),
        ],
    )
    def bmu(x_hbm, wt_hbm, out_hbm, x_v, wt_v, row_v):
        s = lax.axis_index("s")

        @pl.when(s < B)
        def _():
            pltpu.sync_copy(x_hbm.at[pl.ds(s * D, D)], x_v)
            pltpu.sync_copy(wt_hbm, wt_v)

            iota = lax.iota(jnp.int32, L)

            def body(d, accs):
                xd = plsc.load_gather(x_v, [jnp.full((L,), d, jnp.int32)])
                new = []
                for cc in range(nchunk):
                    wv = wt_v[pl.ds(d * N + cc * L, L)]
                    df = xd - wv
                    new.append(accs[cc] + df * df)
                return tuple(new)

            accs = lax.fori_loop(
                0, D, body,
                tuple(jnp.zeros((L,), jnp.float32) for _ in range(nchunk)))

            best_val = accs[0]
            best_idx = iota
            for cc in range(1, nchunk):
                m = accs[cc] < best_val
                best_val = jnp.where(m, accs[cc], best_val)
                best_idx = jnp.where(m, cc * L + iota, best_idx)
            mval = jnp.min(best_val)
            cand = jnp.where(best_val == mval, best_idx, jnp.int32(2**30))
            j = jnp.min(cand)

            for k in range(D // L):
                row_v[pl.ds(k * L, L)] = plsc.load_gather(
                    wt_v, [(iota + k * L) * N + j])
            pltpu.sync_copy(row_v, out_hbm.at[pl.ds(s * D, D)])

    return bmu


def kernel(x, weights):
    b, _, f = x.shape
    n = weights.shape[0]
    xq = x[:, -1, :].reshape(b * f)
    wt = weights.T.reshape(f * n)
    out = _make_bmu_kernel(b, n, f)(xq, wt)
    return out.reshape(b, f)
